# baseline (speedup 1.0000x reference)
import functools

import jax
import jax.numpy as jnp
from jax import lax
from jax.experimental import pallas as pl
from jax.experimental.pallas import tpu as pltpu
from jax.experimental.pallas import tpu_sc as plsc

NC = 2
NS = 16
NW = NC * NS
LANES = 16


N_IN = 2
N_OUT = 2


def _blend_call(pe2, rel2, alpha_vec, seq_len, d, rel_row0):
    rows_per_w = seq_len // NW
    ch_rows = min(8, rows_per_w)
    n_ch = rows_per_w // ch_rows

    mesh = plsc.VectorSubcoreMesh(
        core_axis_name="c", subcore_axis_name="s",
        num_cores=NC, num_subcores=NS)

    buf = pltpu.VMEM((ch_rows, d), jnp.float32)
    sem = pltpu.SemaphoreType.DMA

    @functools.partial(
        pl.kernel,
        out_type=jax.ShapeDtypeStruct((seq_len, d), jnp.float32),
        mesh=mesh,
        compiler_params=pltpu.CompilerParams(use_tc_tiling_on_sc=True),
        scratch_types=(
            [pltpu.VMEM((LANES,), jnp.float32)]
            + [buf] * (2 * N_IN + N_OUT)
            + [sem] * (N_IN + N_OUT)
        ),
    )
    def run(pe_hbm, rel_hbm, al_hbm, out_hbm, al_v, *bufs_and_sems):
        pe_bufs = list(bufs_and_sems[0:N_IN])
        rel_bufs = list(bufs_and_sems[N_IN:2 * N_IN])
        out_bufs = list(bufs_and_sems[2 * N_IN:2 * N_IN + N_OUT])
        s_in = list(bufs_and_sems[2 * N_IN + N_OUT:2 * N_IN + N_OUT + N_IN])
        s_out = list(bufs_and_sems[2 * N_IN + N_OUT + N_IN:])

        wid = lax.axis_index("s") * NC + lax.axis_index("c")
        base_row = wid * rows_per_w
        pltpu.sync_copy(al_hbm, al_v)
        a = al_v[...]
        b = 1.0 - a
        n_vec = d // LANES

        def in_srcs(c):
            row = base_row + c * ch_rows
            return (pe_hbm.at[pl.ds(row, ch_rows), :],
                    rel_hbm.at[pl.ds(rel_row0 + row, ch_rows), :])

        def out_dst(c):
            row = base_row + c * ch_rows
            return out_hbm.at[pl.ds(row, ch_rows), :]

        def start_in(c, slot):
            ps, rs = in_srcs(c)
            pltpu.async_copy(ps, pe_bufs[slot], s_in[slot])
            pltpu.async_copy(rs, rel_bufs[slot], s_in[slot])

        for c in range(min(N_IN - 1, n_ch)):
            start_in(c, c % N_IN)

        n_super = n_ch // N_IN

        def super_body(sp, _):
            for k in range(N_IN):
                c = sp * N_IN + k
                s = k % N_IN
                nxt = c + N_IN - 1

                @pl.when(nxt < n_ch)
                def _():
                    start_in(nxt, (k + N_IN - 1) % N_IN)

                psc, rsc = in_srcs(c)
                pltpu.make_async_copy(psc, pe_bufs[s], s_in[s]).wait()
                pltpu.make_async_copy(rsc, rel_bufs[s], s_in[s]).wait()
                so = k % N_OUT

                @pl.when(c >= N_OUT)
                def _():
                    pltpu.make_async_copy(out_bufs[so], out_dst(c - N_OUT),
                                          s_out[so]).wait()

                pv, rv, ov = pe_bufs[s], rel_bufs[s], out_bufs[so]

                @plsc.parallel_loop(0, ch_rows * n_vec, 1, unroll=4)
                def _(i):
                    r = i // n_vec
                    sl = pl.ds((i % n_vec) * LANES, LANES)
                    ov[r, sl] = a * pv[r, sl] + b * rv[r, sl]

                pltpu.async_copy(ov, out_dst(c), s_out[so])
            return ()

        lax.fori_loop(0, n_super, super_body, ())

        for c in range(max(0, n_ch - N_OUT), n_ch):
            so = (c % N_IN) % N_OUT
            pltpu.make_async_copy(out_bufs[so], out_dst(c), s_out[so]).wait()

    return run(pe2, rel2, alpha_vec)


def kernel(x, pe, rel_table, alpha):
    seq_len = x.shape[1]
    d = pe.shape[-1]
    rel_row0 = pe.shape[1] - seq_len

    pe2 = pe.reshape(pe.shape[1], d)
    alpha_vec = jnp.full((LANES,), alpha, dtype=jnp.float32)

    out = _blend_call(pe2, rel_table, alpha_vec, seq_len, d, rel_row0)
    return out.reshape(1, seq_len, d)


# speedup vs baseline: 1.0047x; 1.0047x over previous
import functools

import jax
import jax.numpy as jnp
from jax import lax
from jax.experimental import pallas as pl
from jax.experimental.pallas import tpu as pltpu
from jax.experimental.pallas import tpu_sc as plsc

NC = 2
NS = 16
NW = NC * NS
LANES = 16


N_IN = 2
N_OUT = 2


def _blend_call(pe2, rel2, alpha_vec, seq_len, d, rel_row0):
    rows_per_w = seq_len // NW
    ch_rows = min(16, rows_per_w)
    n_ch = rows_per_w // ch_rows

    mesh = plsc.VectorSubcoreMesh(
        core_axis_name="c", subcore_axis_name="s",
        num_cores=NC, num_subcores=NS)

    buf = pltpu.VMEM((ch_rows, d), jnp.float32)
    sem = pltpu.SemaphoreType.DMA

    @functools.partial(
        pl.kernel,
        out_type=jax.ShapeDtypeStruct((seq_len, d), jnp.float32),
        mesh=mesh,
        compiler_params=pltpu.CompilerParams(use_tc_tiling_on_sc=True),
        scratch_types=(
            [pltpu.VMEM((LANES,), jnp.float32)]
            + [buf] * (2 * N_IN + N_OUT)
            + [sem] * (N_IN + N_OUT)
        ),
    )
    def run(pe_hbm, rel_hbm, al_hbm, out_hbm, al_v, *bufs_and_sems):
        pe_bufs = list(bufs_and_sems[0:N_IN])
        rel_bufs = list(bufs_and_sems[N_IN:2 * N_IN])
        out_bufs = list(bufs_and_sems[2 * N_IN:2 * N_IN + N_OUT])
        s_in = list(bufs_and_sems[2 * N_IN + N_OUT:2 * N_IN + N_OUT + N_IN])
        s_out = list(bufs_and_sems[2 * N_IN + N_OUT + N_IN:])

        wid = lax.axis_index("s") * NC + lax.axis_index("c")
        base_row = wid * rows_per_w
        pltpu.sync_copy(al_hbm, al_v)
        a = al_v[...]
        b = 1.0 - a
        n_vec = d // LANES

        def in_srcs(c):
            row = base_row + c * ch_rows
            return (pe_hbm.at[pl.ds(row, ch_rows), :],
                    rel_hbm.at[pl.ds(rel_row0 + row, ch_rows), :])

        def out_dst(c):
            row = base_row + c * ch_rows
            return out_hbm.at[pl.ds(row, ch_rows), :]

        def start_in(c, slot):
            ps, rs = in_srcs(c)
            pltpu.async_copy(ps, pe_bufs[slot], s_in[slot])
            pltpu.async_copy(rs, rel_bufs[slot], s_in[slot])

        for c in range(min(N_IN - 1, n_ch)):
            start_in(c, c % N_IN)

        n_super = n_ch // N_IN

        def super_body(sp, _):
            for k in range(N_IN):
                c = sp * N_IN + k
                s = k % N_IN
                nxt = c + N_IN - 1

                @pl.when(nxt < n_ch)
                def _():
                    start_in(nxt, (k + N_IN - 1) % N_IN)

                psc, rsc = in_srcs(c)
                pltpu.make_async_copy(psc, pe_bufs[s], s_in[s]).wait()
                pltpu.make_async_copy(rsc, rel_bufs[s], s_in[s]).wait()
                so = k % N_OUT

                @pl.when(c >= N_OUT)
                def _():
                    pltpu.make_async_copy(out_bufs[so], out_dst(c - N_OUT),
                                          s_out[so]).wait()

                pv, rv, ov = pe_bufs[s], rel_bufs[s], out_bufs[so]

                @plsc.parallel_loop(0, ch_rows * n_vec, 1, unroll=8)
                def _(i):
                    r = i // n_vec
                    sl = pl.ds((i % n_vec) * LANES, LANES)
                    ov[r, sl] = a * pv[r, sl] + b * rv[r, sl]

                pltpu.async_copy(ov, out_dst(c), s_out[so])
            return ()

        lax.fori_loop(0, n_super, super_body, ())

        for c in range(max(0, n_ch - N_OUT), n_ch):
            so = (c % N_IN) % N_OUT
            pltpu.make_async_copy(out_bufs[so], out_dst(c), s_out[so]).wait()

    return run(pe2, rel2, alpha_vec)


def kernel(x, pe, rel_table, alpha):
    seq_len = x.shape[1]
    d = pe.shape[-1]
    rel_row0 = pe.shape[1] - seq_len

    pe2 = pe.reshape(pe.shape[1], d)
    alpha_vec = jnp.full((LANES,), alpha, dtype=jnp.float32)

    out = _blend_call(pe2, rel_table, alpha_vec, seq_len, d, rel_row0)
    return out.reshape(1, seq_len, d)


# speedup vs baseline: 1.0291x; 1.0244x over previous
import functools

import jax
import jax.numpy as jnp
from jax import lax
from jax.experimental import pallas as pl
from jax.experimental.pallas import tpu as pltpu
from jax.experimental.pallas import tpu_sc as plsc

NC = 2
NS = 16
NW = NC * NS
LANES = 16


N_IN = 4
N_OUT = 2


def _blend_call(pe2, rel2, alpha_vec, seq_len, d, rel_row0):
    rows_per_w = seq_len // NW
    ch_rows = min(8, rows_per_w)
    n_ch = rows_per_w // ch_rows

    mesh = plsc.VectorSubcoreMesh(
        core_axis_name="c", subcore_axis_name="s",
        num_cores=NC, num_subcores=NS)

    buf = pltpu.VMEM((ch_rows, d), jnp.float32)
    sem = pltpu.SemaphoreType.DMA

    @functools.partial(
        pl.kernel,
        out_type=jax.ShapeDtypeStruct((seq_len, d), jnp.float32),
        mesh=mesh,
        compiler_params=pltpu.CompilerParams(use_tc_tiling_on_sc=True),
        scratch_types=(
            [pltpu.VMEM((LANES,), jnp.float32)]
            + [buf] * (2 * N_IN + N_OUT)
            + [sem] * (N_IN + N_OUT)
        ),
    )
    def run(pe_hbm, rel_hbm, al_hbm, out_hbm, al_v, *bufs_and_sems):
        pe_bufs = list(bufs_and_sems[0:N_IN])
        rel_bufs = list(bufs_and_sems[N_IN:2 * N_IN])
        out_bufs = list(bufs_and_sems[2 * N_IN:2 * N_IN + N_OUT])
        s_in = list(bufs_and_sems[2 * N_IN + N_OUT:2 * N_IN + N_OUT + N_IN])
        s_out = list(bufs_and_sems[2 * N_IN + N_OUT + N_IN:])

        wid = lax.axis_index("s") * NC + lax.axis_index("c")
        base_row = wid * rows_per_w
        pltpu.sync_copy(al_hbm, al_v)
        a = al_v[...]
        b = 1.0 - a
        n_vec = d // LANES

        def in_srcs(c):
            row = base_row + c * ch_rows
            return (pe_hbm.at[pl.ds(row, ch_rows), :],
                    rel_hbm.at[pl.ds(rel_row0 + row, ch_rows), :])

        def out_dst(c):
            row = base_row + c * ch_rows
            return out_hbm.at[pl.ds(row, ch_rows), :]

        def start_in(c, slot):
            ps, rs = in_srcs(c)
            pltpu.async_copy(ps, pe_bufs[slot], s_in[slot])
            pltpu.async_copy(rs, rel_bufs[slot], s_in[slot])

        for c in range(min(N_IN - 1, n_ch)):
            start_in(c, c % N_IN)

        n_super = n_ch // N_IN

        def super_body(sp, _):
            for k in range(N_IN):
                c = sp * N_IN + k
                s = k % N_IN
                nxt = c + N_IN - 1

                @pl.when(nxt < n_ch)
                def _():
                    start_in(nxt, (k + N_IN - 1) % N_IN)

                psc, rsc = in_srcs(c)
                pltpu.make_async_copy(psc, pe_bufs[s], s_in[s]).wait()
                pltpu.make_async_copy(rsc, rel_bufs[s], s_in[s]).wait()
                so = k % N_OUT

                @pl.when(c >= N_OUT)
                def _():
                    pltpu.make_async_copy(out_bufs[so], out_dst(c - N_OUT),
                                          s_out[so]).wait()

                pv, rv, ov = pe_bufs[s], rel_bufs[s], out_bufs[so]

                @plsc.parallel_loop(0, ch_rows * n_vec, 1, unroll=8)
                def _(i):
                    r = i // n_vec
                    sl = pl.ds((i % n_vec) * LANES, LANES)
                    ov[r, sl] = a * pv[r, sl] + b * rv[r, sl]

                pltpu.async_copy(ov, out_dst(c), s_out[so])
            return ()

        lax.fori_loop(0, n_super, super_body, ())

        for c in range(max(0, n_ch - N_OUT), n_ch):
            so = (c % N_IN) % N_OUT
            pltpu.make_async_copy(out_bufs[so], out_dst(c), s_out[so]).wait()

    return run(pe2, rel2, alpha_vec)


def kernel(x, pe, rel_table, alpha):
    seq_len = x.shape[1]
    d = pe.shape[-1]
    rel_row0 = pe.shape[1] - seq_len

    pe2 = pe.reshape(pe.shape[1], d)
    alpha_vec = jnp.full((LANES,), alpha, dtype=jnp.float32)

    out = _blend_call(pe2, rel_table, alpha_vec, seq_len, d, rel_row0)
    return out.reshape(1, seq_len, d)


# speedup vs baseline: 1.0299x; 1.0008x over previous
import functools

import jax
import jax.numpy as jnp
from jax import lax
from jax.experimental import pallas as pl
from jax.experimental.pallas import tpu as pltpu
from jax.experimental.pallas import tpu_sc as plsc

NC = 2
NS = 16
NW = NC * NS
LANES = 16


N_IN = 4
N_OUT = 4


def _blend_call(pe2, rel2, alpha_vec, seq_len, d, rel_row0):
    rows_per_w = seq_len // NW
    ch_rows = min(8, rows_per_w)
    n_ch = rows_per_w // ch_rows

    mesh = plsc.VectorSubcoreMesh(
        core_axis_name="c", subcore_axis_name="s",
        num_cores=NC, num_subcores=NS)

    buf = pltpu.VMEM((ch_rows, d), jnp.float32)
    sem = pltpu.SemaphoreType.DMA

    @functools.partial(
        pl.kernel,
        out_type=jax.ShapeDtypeStruct((seq_len, d), jnp.float32),
        mesh=mesh,
        compiler_params=pltpu.CompilerParams(use_tc_tiling_on_sc=True),
        scratch_types=(
            [pltpu.VMEM((LANES,), jnp.float32)]
            + [buf] * (2 * N_IN + N_OUT)
            + [sem] * (N_IN + N_OUT)
        ),
    )
    def run(pe_hbm, rel_hbm, al_hbm, out_hbm, al_v, *bufs_and_sems):
        pe_bufs = list(bufs_and_sems[0:N_IN])
        rel_bufs = list(bufs_and_sems[N_IN:2 * N_IN])
        out_bufs = list(bufs_and_sems[2 * N_IN:2 * N_IN + N_OUT])
        s_in = list(bufs_and_sems[2 * N_IN + N_OUT:2 * N_IN + N_OUT + N_IN])
        s_out = list(bufs_and_sems[2 * N_IN + N_OUT + N_IN:])

        wid = lax.axis_index("s") * NC + lax.axis_index("c")
        base_row = wid * rows_per_w
        pltpu.sync_copy(al_hbm, al_v)
        a = al_v[...]
        b = 1.0 - a
        n_vec = d // LANES

        def in_srcs(c):
            row = base_row + c * ch_rows
            return (pe_hbm.at[pl.ds(row, ch_rows), :],
                    rel_hbm.at[pl.ds(rel_row0 + row, ch_rows), :])

        def out_dst(c):
            row = base_row + c * ch_rows
            return out_hbm.at[pl.ds(row, ch_rows), :]

        def start_in(c, slot):
            ps, rs = in_srcs(c)
            pltpu.async_copy(ps, pe_bufs[slot], s_in[slot])
            pltpu.async_copy(rs, rel_bufs[slot], s_in[slot])

        for c in range(min(N_IN - 1, n_ch)):
            start_in(c, c % N_IN)

        n_super = n_ch // N_IN

        def super_body(sp, _):
            for k in range(N_IN):
                c = sp * N_IN + k
                s = k % N_IN
                nxt = c + N_IN - 1

                @pl.when(nxt < n_ch)
                def _():
                    start_in(nxt, (k + N_IN - 1) % N_IN)

                psc, rsc = in_srcs(c)
                pltpu.make_async_copy(psc, pe_bufs[s], s_in[s]).wait()
                pltpu.make_async_copy(rsc, rel_bufs[s], s_in[s]).wait()
                so = k % N_OUT

                @pl.when(c >= N_OUT)
                def _():
                    pltpu.make_async_copy(out_bufs[so], out_dst(c - N_OUT),
                                          s_out[so]).wait()

                pv, rv, ov = pe_bufs[s], rel_bufs[s], out_bufs[so]

                @plsc.parallel_loop(0, ch_rows * n_vec, 1, unroll=8)
                def _(i):
                    r = i // n_vec
                    sl = pl.ds((i % n_vec) * LANES, LANES)
                    ov[r, sl] = a * pv[r, sl] + b * rv[r, sl]

                pltpu.async_copy(ov, out_dst(c), s_out[so])
            return ()

        lax.fori_loop(0, n_super, super_body, ())

        for c in range(max(0, n_ch - N_OUT), n_ch):
            so = (c % N_IN) % N_OUT
            pltpu.make_async_copy(out_bufs[so], out_dst(c), s_out[so]).wait()

    return run(pe2, rel2, alpha_vec)


def kernel(x, pe, rel_table, alpha):
    seq_len = x.shape[1]
    d = pe.shape[-1]
    rel_row0 = pe.shape[1] - seq_len

    pe2 = pe.reshape(pe.shape[1], d)
    alpha_vec = jnp.full((LANES,), alpha, dtype=jnp.float32)

    out = _blend_call(pe2, rel_table, alpha_vec, seq_len, d, rel_row0)
    return out.reshape(1, seq_len, d)
